# Initial kernel scaffold; baseline (speedup 1.0000x reference)
import functools

import jax
import jax.numpy as jnp
from jax import lax
from jax.experimental import pallas as pl
from jax.experimental.pallas import tpu as pltpu
from jax.experimental.pallas import tpu_sc as plsc

N_NODES_K = 10000
D_FEAT_K = 128
N_EDGES_K = 320000
CHUNK = 512
N_CHUNKS = N_EDGES_K // CHUNK
N_TILES = 32
ROWS_PER_TILE = N_NODES_K // 16
ZROWS = 125


def _hop_body(table, row, col, trend, out0, out1,
              ridx, cidx, tbuf, rows, zbuf, acc, sem):
    c = lax.axis_index("c")
    s = lax.axis_index("s")
    wid = s * 2 + c

    def _zfill(r, carry):
        for d in range(D_FEAT_K // 16):
            zbuf[r, pl.ds(d * 16, 16)] = jnp.zeros((16,), jnp.float32)
        return carry

    lax.fori_loop(0, ZROWS, _zfill, 0)
    for j in range(ROWS_PER_TILE // ZROWS):
        pltpu.sync_copy(zbuf, acc.at[pl.ds(s * ROWS_PER_TILE + j * ZROWS, ZROWS)])
    plsc.subcore_barrier()

    n_my = (N_CHUNKS - wid + N_TILES - 1) // N_TILES

    def _chunk(i, carry):
        base = (wid + i * N_TILES) * CHUNK
        pltpu.sync_copy(row.at[pl.ds(base, CHUNK)], ridx)
        pltpu.sync_copy(col.at[pl.ds(base, CHUNK)], cidx)
        pltpu.sync_copy(trend.at[pl.ds(base, CHUNK)], tbuf)
        pltpu.async_copy(table.at[ridx], rows, sem).wait()

        def _scale(k, c2):
            t = tbuf[k]
            for d in range(D_FEAT_K // 16):
                sl = pl.ds(d * 16, 16)
                rows[k, sl] = rows[k, sl] * t
            return c2

        lax.fori_loop(0, CHUNK, _scale, 0)
        pltpu.sync_copy(rows, acc.at[cidx], add=True)
        return carry

    lax.fori_loop(0, n_my, _chunk, 0)
    plsc.subcore_barrier()

    sl = pl.ds(s * ROWS_PER_TILE, ROWS_PER_TILE)

    @pl.when(c == 0)
    def _():
        pltpu.sync_copy(acc.at[sl], out0.at[sl])

    @pl.when(c == 1)
    def _():
        pltpu.sync_copy(acc.at[sl], out1.at[sl])


def _sc_hop(table, row, col, trend):
    mesh = plsc.VectorSubcoreMesh(core_axis_name="c", subcore_axis_name="s")
    f = functools.partial(
        pl.kernel,
        mesh=mesh,
        out_type=[
            jax.ShapeDtypeStruct((N_NODES_K, D_FEAT_K), jnp.float32),
            jax.ShapeDtypeStruct((N_NODES_K, D_FEAT_K), jnp.float32),
        ],
        scratch_types=[
            pltpu.VMEM((CHUNK,), jnp.int32),
            pltpu.VMEM((CHUNK,), jnp.int32),
            pltpu.VMEM((CHUNK,), jnp.float32),
            pltpu.VMEM((CHUNK, D_FEAT_K), jnp.float32),
            pltpu.VMEM((ZROWS, D_FEAT_K), jnp.float32),
            pltpu.VMEM_SHARED((N_NODES_K, D_FEAT_K), jnp.float32),
            pltpu.SemaphoreType.DMA,
        ],
    )(_hop_body)
    return f(table, row, col, trend)


def _combine_body(a_ref, b_ref, o_ref):
    o_ref[...] = a_ref[...] + b_ref[...]


def _combine(p0, p1):
    return pl.pallas_call(
        _combine_body,
        out_shape=jax.ShapeDtypeStruct((N_NODES_K, D_FEAT_K), jnp.float32),
        grid=(10,),
        in_specs=[
            pl.BlockSpec((N_NODES_K // 10, D_FEAT_K), lambda i: (i, 0)),
            pl.BlockSpec((N_NODES_K // 10, D_FEAT_K), lambda i: (i, 0)),
        ],
        out_specs=pl.BlockSpec((N_NODES_K // 10, D_FEAT_K), lambda i: (i, 0)),
    )(p0, p1)


def kernel(embed, edge_index, trend):
    row = edge_index[0].astype(jnp.int32)
    col = edge_index[1].astype(jnp.int32)
    embs = [embed]
    t = embed
    for _ in range(3):
        p0, p1 = _sc_hop(t, row, col, trend)
        t = _combine(p0, p1)
        embs.append(t)
    return jnp.stack(embs, axis=1)


# speedup vs baseline: 3.4836x; 3.4836x over previous
import functools

import jax
import jax.numpy as jnp
from jax import lax
from jax.experimental import pallas as pl
from jax.experimental.pallas import tpu as pltpu
from jax.experimental.pallas import tpu_sc as plsc

N_NODES_K = 10000
D_FEAT_K = 128
HALF_N = N_NODES_K // 2
ACC_ROWS = HALF_N + 8
N_EDGES_K = 320000
CHUNK = 512
N_CHUNKS = N_EDGES_K // CHUNK
ZPT = 320
ZPT_LAST = ACC_ROWS - 15 * ZPT
WPT = 312
WPT_LAST = HALF_N - 15 * WPT


def _hop_body(table, row, col, trend, out, ridx, cidx, tbuf, rows, acc, sem):
    c = lax.axis_index("c")
    s = lax.axis_index("s")

    def _zfill(r, carry):
        for d in range(D_FEAT_K // 16):
            rows[r, pl.ds(d * 16, 16)] = jnp.zeros((16,), jnp.float32)
        return carry

    lax.fori_loop(0, ZPT, _zfill, 0)

    @pl.when(s < 15)
    def _():
        pltpu.sync_copy(rows.at[pl.ds(0, ZPT)], acc.at[pl.ds(s * ZPT, ZPT)])

    @pl.when(s == 15)
    def _():
        pltpu.sync_copy(rows.at[pl.ds(0, ZPT_LAST)],
                        acc.at[pl.ds(15 * ZPT, ZPT_LAST)])

    plsc.subcore_barrier()

    n_my = (N_CHUNKS - s + 15) // 16
    cbase = c * HALF_N

    def _chunk(i, carry):
        base = (s + i * 16) * CHUNK
        pltpu.sync_copy(row.at[pl.ds(base, CHUNK)], ridx)
        pltpu.sync_copy(col.at[pl.ds(base, CHUNK)], cidx)
        pltpu.sync_copy(trend.at[pl.ds(base, CHUNK)], tbuf)
        pltpu.async_copy(table.at[ridx], rows, sem).wait()

        def _clamp(g, c2):
            sl = pl.ds(g * 16, 16)
            lv = cidx[sl] - cbase
            ok = jnp.logical_and(lv >= 0, lv < HALF_N)
            cidx[sl] = jnp.where(ok, lv, HALF_N)
            return c2

        lax.fori_loop(0, CHUNK // 16, _clamp, 0)

        def _scale(g, c2):
            tv = tbuf[pl.ds(g * 16, 16)]
            for j in range(16):
                k = g * 16 + j
                t = tv[j]
                for d in range(D_FEAT_K // 16):
                    sl = pl.ds(d * 16, 16)
                    rows[k, sl] = rows[k, sl] * t
            return c2

        lax.fori_loop(0, CHUNK // 16, _scale, 0)
        pltpu.sync_copy(rows, acc.at[cidx], add=True)
        return carry

    lax.fori_loop(0, n_my, _chunk, 0)
    plsc.subcore_barrier()

    @pl.when(s < 15)
    def _():
        pltpu.sync_copy(acc.at[pl.ds(s * WPT, WPT)],
                        out.at[pl.ds(cbase + s * WPT, WPT)])

    @pl.when(s == 15)
    def _():
        pltpu.sync_copy(acc.at[pl.ds(15 * WPT, WPT_LAST)],
                        out.at[pl.ds(cbase + 15 * WPT, WPT_LAST)])


def _sc_hop(table, row, col, trend):
    mesh = plsc.VectorSubcoreMesh(core_axis_name="c", subcore_axis_name="s")
    f = functools.partial(
        pl.kernel,
        mesh=mesh,
        out_type=jax.ShapeDtypeStruct((N_NODES_K, D_FEAT_K), jnp.float32),
        scratch_types=[
            pltpu.VMEM((CHUNK,), jnp.int32),
            pltpu.VMEM((CHUNK,), jnp.int32),
            pltpu.VMEM((CHUNK,), jnp.float32),
            pltpu.VMEM((CHUNK, D_FEAT_K), jnp.float32),
            pltpu.VMEM_SHARED((ACC_ROWS, D_FEAT_K), jnp.float32),
            pltpu.SemaphoreType.DMA,
        ],
    )(_hop_body)
    return f(table, row, col, trend)


def _stack_body(e_ref, a1, a2, a3, o_ref):
    o_ref[:, 0, :] = e_ref[...]
    o_ref[:, 1, :] = a1[...]
    o_ref[:, 2, :] = a2[...]
    o_ref[:, 3, :] = a3[...]


def _assemble(embed, hops):
    blk = N_NODES_K // 10
    spec = pl.BlockSpec((blk, D_FEAT_K), lambda i: (i, 0))
    return pl.pallas_call(
        _stack_body,
        out_shape=jax.ShapeDtypeStruct((N_NODES_K, 4, D_FEAT_K), jnp.float32),
        grid=(10,),
        in_specs=[spec, spec, spec, spec],
        out_specs=pl.BlockSpec((blk, 4, D_FEAT_K), lambda i: (i, 0, 0)),
    )(embed, *hops)


def kernel(embed, edge_index, trend):
    row = edge_index[0].astype(jnp.int32)
    col = edge_index[1].astype(jnp.int32)
    t = embed
    hops = []
    for _ in range(3):
        t = _sc_hop(t, row, col, trend)
        hops.append(t)
    return _assemble(embed, hops)


# speedup vs baseline: 3.7471x; 1.0757x over previous
import functools

import jax
import jax.numpy as jnp
from jax import lax
from jax.experimental import pallas as pl
from jax.experimental.pallas import tpu as pltpu
from jax.experimental.pallas import tpu_sc as plsc

N_NODES_K = 10000
D_FEAT_K = 128
HALF_N = N_NODES_K // 2
ACC_ROWS = HALF_N + 8
N_EDGES_K = 320000
CHUNK = 320
N_CHUNKS = N_EDGES_K // CHUNK
ZPT = 320
ZPT_LAST = ACC_ROWS - 15 * ZPT
WPT = 312
WPT_LAST = HALF_N - 15 * WPT


def _hop_body(table, row, col, trend, out,
              ridx0, cidx0, tbuf0, rows0,
              ridx1, cidx1, tbuf1, rows1,
              acc, isem0, isem1, gsem0, gsem1):
    c = lax.axis_index("c")
    s = lax.axis_index("s")

    def _zfill(r, carry):
        for d in range(D_FEAT_K // 16):
            rows0[r, pl.ds(d * 16, 16)] = jnp.zeros((16,), jnp.float32)
        return carry

    lax.fori_loop(0, ZPT, _zfill, 0)

    @pl.when(s < 15)
    def _():
        pltpu.sync_copy(rows0.at[pl.ds(0, ZPT)], acc.at[pl.ds(s * ZPT, ZPT)])

    @pl.when(s == 15)
    def _():
        pltpu.sync_copy(rows0.at[pl.ds(0, ZPT_LAST)],
                        acc.at[pl.ds(15 * ZPT, ZPT_LAST)])

    plsc.subcore_barrier()

    cbase = c * HALF_N
    start = s * 62 + jnp.minimum(s, 8)
    n_my = jnp.where(s < 8, 63, 62)

    bufs = ((ridx0, cidx0, tbuf0, rows0, isem0, gsem0),
            (ridx1, cidx1, tbuf1, rows1, isem1, gsem1))

    def _fetch_idx(j, b):
        ridx, cidx, tbuf, _, isem, _ = bufs[b]

        @pl.when(j < n_my)
        def _():
            base = (start + j) * CHUNK
            pltpu.make_async_copy(row.at[pl.ds(base, CHUNK)], ridx, isem).start()
            pltpu.make_async_copy(col.at[pl.ds(base, CHUNK)], cidx, isem).start()
            pltpu.make_async_copy(trend.at[pl.ds(base, CHUNK)], tbuf, isem).start()

    def _start_gather(j, b):
        ridx, cidx, tbuf, rows, isem, gsem = bufs[b]

        @pl.when(j < n_my)
        def _():
            pltpu.make_async_copy(row.at[pl.ds(0, CHUNK)], ridx, isem).wait()
            pltpu.make_async_copy(col.at[pl.ds(0, CHUNK)], cidx, isem).wait()
            pltpu.make_async_copy(trend.at[pl.ds(0, CHUNK)], tbuf, isem).wait()
            pltpu.make_async_copy(table.at[ridx], rows, gsem).start()

    def _process(j, b):
        ridx, cidx, tbuf, rows, isem, gsem = bufs[b]

        @pl.when(j < n_my)
        def _():
            pltpu.make_async_copy(table.at[ridx], rows, gsem).wait()

            def _scale(g, c2):
                sl16 = pl.ds(g * 16, 16)
                lv = cidx[sl16] - cbase
                ok = jnp.logical_and(lv >= 0, lv < HALF_N)
                cidx[sl16] = jnp.where(ok, lv, HALF_N)
                tv = tbuf[sl16]
                for j2 in range(16):
                    k = g * 16 + j2
                    t = tv[j2]
                    for d in range(D_FEAT_K // 16):
                        sl = pl.ds(d * 16, 16)
                        rows[k, sl] = rows[k, sl] * t
                return c2

            lax.fori_loop(0, CHUNK // 16, _scale, 0)
            pltpu.sync_copy(rows, acc.at[cidx], add=True)

    _fetch_idx(0, 0)
    _start_gather(0, 0)
    _fetch_idx(1, 1)

    def _outer(io, carry):
        for b in range(2):
            j = io * 2 + b
            _start_gather(j + 1, 1 - b)
            _process(j, b)
            _fetch_idx(j + 2, b)
        return carry

    lax.fori_loop(0, 32, _outer, 0)
    plsc.subcore_barrier()

    @pl.when(s < 15)
    def _():
        pltpu.sync_copy(acc.at[pl.ds(s * WPT, WPT)],
                        out.at[pl.ds(cbase + s * WPT, WPT)])

    @pl.when(s == 15)
    def _():
        pltpu.sync_copy(acc.at[pl.ds(15 * WPT, WPT_LAST)],
                        out.at[pl.ds(cbase + 15 * WPT, WPT_LAST)])


def _sc_hop(table, row, col, trend):
    mesh = plsc.VectorSubcoreMesh(core_axis_name="c", subcore_axis_name="s")
    f = functools.partial(
        pl.kernel,
        mesh=mesh,
        out_type=jax.ShapeDtypeStruct((N_NODES_K, D_FEAT_K), jnp.float32),
        scratch_types=[
            pltpu.VMEM((CHUNK,), jnp.int32),
            pltpu.VMEM((CHUNK,), jnp.int32),
            pltpu.VMEM((CHUNK,), jnp.float32),
            pltpu.VMEM((CHUNK, D_FEAT_K), jnp.float32),
            pltpu.VMEM((CHUNK,), jnp.int32),
            pltpu.VMEM((CHUNK,), jnp.int32),
            pltpu.VMEM((CHUNK,), jnp.float32),
            pltpu.VMEM((CHUNK, D_FEAT_K), jnp.float32),
            pltpu.VMEM_SHARED((ACC_ROWS, D_FEAT_K), jnp.float32),
            pltpu.SemaphoreType.DMA,
            pltpu.SemaphoreType.DMA,
            pltpu.SemaphoreType.DMA,
            pltpu.SemaphoreType.DMA,
        ],
    )(_hop_body)
    return f(table, row, col, trend)


def _stack_body(e_ref, a1, a2, a3, o_ref):
    o_ref[:, 0, :] = e_ref[...]
    o_ref[:, 1, :] = a1[...]
    o_ref[:, 2, :] = a2[...]
    o_ref[:, 3, :] = a3[...]


def _assemble(embed, hops):
    blk = N_NODES_K // 10
    spec = pl.BlockSpec((blk, D_FEAT_K), lambda i: (i, 0))
    return pl.pallas_call(
        _stack_body,
        out_shape=jax.ShapeDtypeStruct((N_NODES_K, 4, D_FEAT_K), jnp.float32),
        grid=(10,),
        in_specs=[spec, spec, spec, spec],
        out_specs=pl.BlockSpec((blk, 4, D_FEAT_K), lambda i: (i, 0, 0)),
    )(embed, *hops)


def kernel(embed, edge_index, trend):
    row = edge_index[0].astype(jnp.int32)
    col = edge_index[1].astype(jnp.int32)
    t = embed
    hops = []
    for _ in range(3):
        t = _sc_hop(t, row, col, trend)
        hops.append(t)
    return _assemble(embed, hops)


# speedup vs baseline: 4.8815x; 1.3027x over previous
import functools

import jax
import jax.numpy as jnp
from jax import lax
from jax.experimental import pallas as pl
from jax.experimental.pallas import tpu as pltpu
from jax.experimental.pallas import tpu_sc as plsc

N_NODES_K = 10000
D_FEAT_K = 128
HALF_N = N_NODES_K // 2
ACC_ROWS = HALF_N + 8
N_EDGES_K = 320000
CHUNK = 160
N_CHUNKS = N_EDGES_K // CHUNK
ZPT = 320
ZPT_LAST = ACC_ROWS - 15 * ZPT
WPT = 312
WPT_LAST = HALF_N - 15 * WPT


def _hop_body(table, row, col, trend, out,
              ridx0, cidx0, tbuf0, rows0,
              ridx1, cidx1, tbuf1, rows1,
              ridx2, cidx2, tbuf2, rows2,
              acc, isem0, isem1, isem2,
              gsem0, gsem1, gsem2, ssem0, ssem1, ssem2):
    c = lax.axis_index("c")
    s = lax.axis_index("s")

    def _zfill(r, carry):
        for d in range(D_FEAT_K // 16):
            rows0[r, pl.ds(d * 16, 16)] = jnp.zeros((16,), jnp.float32)
        return carry

    lax.fori_loop(0, CHUNK, _zfill, 0)

    @pl.when(s < 15)
    def _():
        pltpu.sync_copy(rows0, acc.at[pl.ds(s * ZPT, CHUNK)])
        pltpu.sync_copy(rows0, acc.at[pl.ds(s * ZPT + CHUNK, CHUNK)])

    @pl.when(s == 15)
    def _():
        pltpu.sync_copy(rows0, acc.at[pl.ds(15 * ZPT, CHUNK)])
        pltpu.sync_copy(rows0.at[pl.ds(0, ZPT_LAST - CHUNK)],
                        acc.at[pl.ds(15 * ZPT + CHUNK, ZPT_LAST - CHUNK)])

    plsc.subcore_barrier()

    cbase = c * HALF_N
    per = N_CHUNKS // 16
    rem = N_CHUNKS - 16 * per
    start = s * per + jnp.minimum(s, rem)
    n_my = jnp.where(s < rem, per + 1, per)

    bufs = ((ridx0, cidx0, tbuf0, rows0, isem0, gsem0, ssem0),
            (ridx1, cidx1, tbuf1, rows1, isem1, gsem1, ssem1),
            (ridx2, cidx2, tbuf2, rows2, isem2, gsem2, ssem2))

    def _fetch_idx(j, b):
        ridx, cidx, tbuf, _, isem, _, _ = bufs[b]

        @pl.when(j < n_my)
        def _():
            base = (start + j) * CHUNK
            pltpu.make_async_copy(row.at[pl.ds(base, CHUNK)], ridx, isem).start()
            pltpu.make_async_copy(col.at[pl.ds(base, CHUNK)], cidx, isem).start()
            pltpu.make_async_copy(trend.at[pl.ds(base, CHUNK)], tbuf, isem).start()

    def _start_gather(j, b):
        ridx, cidx, tbuf, rows, isem, gsem, _ = bufs[b]

        @pl.when(j < n_my)
        def _():
            pltpu.make_async_copy(row.at[pl.ds(0, CHUNK)], ridx, isem).wait()
            pltpu.make_async_copy(col.at[pl.ds(0, CHUNK)], cidx, isem).wait()
            pltpu.make_async_copy(trend.at[pl.ds(0, CHUNK)], tbuf, isem).wait()
            pltpu.make_async_copy(table.at[ridx], rows, gsem).start()

    def _wait_gather_scale(j, b):
        ridx, cidx, tbuf, rows, isem, gsem, _ = bufs[b]

        @pl.when(j < n_my)
        def _():
            pltpu.make_async_copy(table.at[ridx], rows, gsem).wait()

            def _scale(g, c2):
                sl16 = pl.ds(g * 16, 16)
                lv = cidx[sl16] - cbase
                ok = jnp.logical_and(lv >= 0, lv < HALF_N)
                cidx[sl16] = jnp.where(ok, lv, HALF_N)
                tv = tbuf[sl16]
                for j2 in range(16):
                    k = g * 16 + j2
                    t = tv[j2]
                    for d in range(D_FEAT_K // 16):
                        sl = pl.ds(d * 16, 16)
                        rows[k, sl] = rows[k, sl] * t
                return c2

            lax.fori_loop(0, CHUNK // 16, _scale, 0)

    def _start_scatter(j, b):
        _, cidx, _, rows, _, _, ssem = bufs[b]

        @pl.when(j < n_my)
        def _():
            pltpu.async_copy(rows, acc.at[cidx], ssem, add=True)

    def _wait_scatter(j, b):
        _, cidx, _, rows, _, _, ssem = bufs[b]

        @pl.when(jnp.logical_and(j >= 0, j < n_my))
        def _():
            pltpu.make_async_copy(rows, acc.at[cidx], ssem).wait()

    _fetch_idx(0, 0)
    _start_gather(0, 0)
    _fetch_idx(1, 1)

    def _outer(io, carry):
        for b in range(3):
            j = io * 3 + b
            nb = (b + 1) % 3
            pb = (b + 2) % 3
            _start_gather(j + 1, nb)
            _wait_gather_scale(j, b)
            _wait_scatter(j - 1, pb)
            _fetch_idx(j + 2, pb)
            _start_scatter(j, b)
        return carry

    lax.fori_loop(0, (per + 3) // 3 + 1, _outer, 0)
    plsc.subcore_barrier()

    @pl.when(s < 15)
    def _():
        pltpu.sync_copy(acc.at[pl.ds(s * WPT, WPT)],
                        out.at[pl.ds(cbase + s * WPT, WPT)])

    @pl.when(s == 15)
    def _():
        pltpu.sync_copy(acc.at[pl.ds(15 * WPT, WPT_LAST)],
                        out.at[pl.ds(cbase + 15 * WPT, WPT_LAST)])


def _sc_hop(table, row, col, trend):
    mesh = plsc.VectorSubcoreMesh(core_axis_name="c", subcore_axis_name="s")
    f = functools.partial(
        pl.kernel,
        mesh=mesh,
        out_type=jax.ShapeDtypeStruct((N_NODES_K, D_FEAT_K), jnp.float32),
        scratch_types=(
            [pltpu.VMEM((CHUNK,), jnp.int32),
             pltpu.VMEM((CHUNK,), jnp.int32),
             pltpu.VMEM((CHUNK,), jnp.float32),
             pltpu.VMEM((CHUNK, D_FEAT_K), jnp.float32)] * 3
            + [pltpu.VMEM_SHARED((ACC_ROWS, D_FEAT_K), jnp.float32)]
            + [pltpu.SemaphoreType.DMA] * 9
        ),
    )(_hop_body)
    return f(table, row, col, trend)


def _stack_body(e_ref, a1, a2, a3, o_ref):
    o_ref[:, 0, :] = e_ref[...]
    o_ref[:, 1, :] = a1[...]
    o_ref[:, 2, :] = a2[...]
    o_ref[:, 3, :] = a3[...]


def _assemble(embed, hops):
    blk = N_NODES_K // 10
    spec = pl.BlockSpec((blk, D_FEAT_K), lambda i: (i, 0))
    return pl.pallas_call(
        _stack_body,
        out_shape=jax.ShapeDtypeStruct((N_NODES_K, 4, D_FEAT_K), jnp.float32),
        grid=(10,),
        in_specs=[spec, spec, spec, spec],
        out_specs=pl.BlockSpec((blk, 4, D_FEAT_K), lambda i: (i, 0, 0)),
    )(embed, *hops)


def kernel(embed, edge_index, trend):
    row = edge_index[0].astype(jnp.int32)
    col = edge_index[1].astype(jnp.int32)
    t = embed
    hops = []
    for _ in range(3):
        t = _sc_hop(t, row, col, trend)
        hops.append(t)
    return _assemble(embed, hops)


# speedup vs baseline: 10.2201x; 2.0937x over previous
import functools

import jax
import jax.numpy as jnp
from jax import lax
from jax.experimental import pallas as pl
from jax.experimental.pallas import tpu as pltpu
from jax.experimental.pallas import tpu_sc as plsc

N_NODES_K = 10000
D_FEAT_K = 128
N_EDGES_K = 320000
CHUNK = 128
N_CHUNKS = N_EDGES_K // CHUNK
SC_CHUNKS = N_CHUNKS // 2
RPT = 632
RPT_LAST = N_NODES_K - 15 * RPT


def _hop_body(table, row, col, trend, out0, out1,
              ridx0, cidx0, tbuf0, rows0,
              ridx1, cidx1, tbuf1, rows1,
              ridx2, cidx2, tbuf2, rows2,
              acc, isem0, isem1, isem2,
              gsem0, gsem1, gsem2, ssem0, ssem1, ssem2):
    c = lax.axis_index("c")
    s = lax.axis_index("s")

    def _zfill(r, carry):
        for d in range(D_FEAT_K // 16):
            rows0[r, pl.ds(d * 16, 16)] = jnp.zeros((16,), jnp.float32)
        return carry

    lax.fori_loop(0, CHUNK, _zfill, 0)

    @pl.when(s < 15)
    def _():
        for q in range(4):
            pltpu.sync_copy(rows0, acc.at[pl.ds(s * RPT + q * CHUNK, CHUNK)])
        pltpu.sync_copy(rows0.at[pl.ds(0, RPT - 4 * CHUNK)],
                        acc.at[pl.ds(s * RPT + 4 * CHUNK, RPT - 4 * CHUNK)])

    @pl.when(s == 15)
    def _():
        for q in range(4):
            pltpu.sync_copy(rows0, acc.at[pl.ds(15 * RPT + q * CHUNK, CHUNK)])
        pltpu.sync_copy(rows0.at[pl.ds(0, RPT_LAST - 4 * CHUNK)],
                        acc.at[pl.ds(15 * RPT + 4 * CHUNK, RPT_LAST - 4 * CHUNK)])

    plsc.subcore_barrier()

    per = SC_CHUNKS // 16
    rem = SC_CHUNKS - 16 * per
    start = c * SC_CHUNKS + s * per + jnp.minimum(s, rem)
    n_my = jnp.where(s < rem, per + 1, per)

    bufs = ((ridx0, cidx0, tbuf0, rows0, isem0, gsem0, ssem0),
            (ridx1, cidx1, tbuf1, rows1, isem1, gsem1, ssem1),
            (ridx2, cidx2, tbuf2, rows2, isem2, gsem2, ssem2))

    def _fetch_idx(j, b):
        ridx, cidx, tbuf, _, isem, _, _ = bufs[b]

        @pl.when(j < n_my)
        def _():
            base = (start + j) * CHUNK
            pltpu.make_async_copy(row.at[pl.ds(base, CHUNK)], ridx, isem).start()
            pltpu.make_async_copy(col.at[pl.ds(base, CHUNK)], cidx, isem).start()
            pltpu.make_async_copy(trend.at[pl.ds(base, CHUNK)], tbuf, isem).start()

    def _start_gather(j, b):
        ridx, cidx, tbuf, rows, isem, gsem, _ = bufs[b]

        @pl.when(j < n_my)
        def _():
            pltpu.make_async_copy(row.at[pl.ds(0, CHUNK)], ridx, isem).wait()
            pltpu.make_async_copy(col.at[pl.ds(0, CHUNK)], cidx, isem).wait()
            pltpu.make_async_copy(trend.at[pl.ds(0, CHUNK)], tbuf, isem).wait()
            pltpu.make_async_copy(table.at[ridx], rows, gsem).start()

    def _wait_gather_scale(j, b):
        ridx, cidx, tbuf, rows, isem, gsem, _ = bufs[b]

        @pl.when(j < n_my)
        def _():
            pltpu.make_async_copy(table.at[ridx], rows, gsem).wait()

            def _scale(g, c2):
                sl16 = pl.ds(g * 16, 16)
                tv = tbuf[sl16]
                for j2 in range(16):
                    k = g * 16 + j2
                    t = tv[j2]
                    for d in range(D_FEAT_K // 16):
                        sl = pl.ds(d * 16, 16)
                        rows[k, sl] = rows[k, sl] * t
                return c2

            lax.fori_loop(0, CHUNK // 16, _scale, 0)

    def _start_scatter(j, b):
        _, cidx, _, rows, _, _, ssem = bufs[b]

        @pl.when(j < n_my)
        def _():
            pltpu.async_copy(rows, acc.at[cidx], ssem, add=True)

    def _wait_scatter(j, b):
        _, cidx, _, rows, _, _, ssem = bufs[b]

        @pl.when(jnp.logical_and(j >= 0, j < n_my))
        def _():
            pltpu.make_async_copy(rows, acc.at[cidx], ssem).wait()

    _fetch_idx(0, 0)
    _start_gather(0, 0)
    _fetch_idx(1, 1)

    def _outer(io, carry):
        for b in range(3):
            j = io * 3 + b
            nb = (b + 1) % 3
            pb = (b + 2) % 3
            _start_gather(j + 1, nb)
            _wait_gather_scale(j, b)
            _wait_scatter(j - 1, pb)
            _fetch_idx(j + 2, pb)
            _start_scatter(j, b)
        return carry

    lax.fori_loop(0, (per + 3) // 3 + 1, _outer, 0)
    plsc.subcore_barrier()

    @pl.when(jnp.logical_and(c == 0, s < 15))
    def _():
        sl = pl.ds(s * RPT, RPT)
        pltpu.sync_copy(acc.at[sl], out0.at[sl])

    @pl.when(jnp.logical_and(c == 0, s == 15))
    def _():
        sl = pl.ds(15 * RPT, RPT_LAST)
        pltpu.sync_copy(acc.at[sl], out0.at[sl])

    @pl.when(jnp.logical_and(c == 1, s < 15))
    def _():
        sl = pl.ds(s * RPT, RPT)
        pltpu.sync_copy(acc.at[sl], out1.at[sl])

    @pl.when(jnp.logical_and(c == 1, s == 15))
    def _():
        sl = pl.ds(15 * RPT, RPT_LAST)
        pltpu.sync_copy(acc.at[sl], out1.at[sl])


def _sc_hop(table, row, col, trend):
    mesh = plsc.VectorSubcoreMesh(core_axis_name="c", subcore_axis_name="s")
    f = functools.partial(
        pl.kernel,
        mesh=mesh,
        out_type=[
            jax.ShapeDtypeStruct((N_NODES_K, D_FEAT_K), jnp.float32),
            jax.ShapeDtypeStruct((N_NODES_K, D_FEAT_K), jnp.float32),
        ],
        scratch_types=(
            [pltpu.VMEM((CHUNK,), jnp.int32),
             pltpu.VMEM((CHUNK,), jnp.int32),
             pltpu.VMEM((CHUNK,), jnp.float32),
             pltpu.VMEM((CHUNK, D_FEAT_K), jnp.float32)] * 3
            + [pltpu.VMEM_SHARED((N_NODES_K, D_FEAT_K), jnp.float32)]
            + [pltpu.SemaphoreType.DMA] * 9
        ),
    )(_hop_body)
    return f(table, row, col, trend)


def _combine_body(a_ref, b_ref, o_ref):
    o_ref[...] = a_ref[...] + b_ref[...]


def _combine(p0, p1):
    blk = N_NODES_K // 10
    spec = pl.BlockSpec((blk, D_FEAT_K), lambda i: (i, 0))
    return pl.pallas_call(
        _combine_body,
        out_shape=jax.ShapeDtypeStruct((N_NODES_K, D_FEAT_K), jnp.float32),
        grid=(10,),
        in_specs=[spec, spec],
        out_specs=spec,
    )(p0, p1)


def _stack_body(e_ref, a1, a2, a3, o_ref):
    o_ref[:, 0, :] = e_ref[...]
    o_ref[:, 1, :] = a1[...]
    o_ref[:, 2, :] = a2[...]
    o_ref[:, 3, :] = a3[...]


def _assemble(embed, hops):
    blk = N_NODES_K // 10
    spec = pl.BlockSpec((blk, D_FEAT_K), lambda i: (i, 0))
    return pl.pallas_call(
        _stack_body,
        out_shape=jax.ShapeDtypeStruct((N_NODES_K, 4, D_FEAT_K), jnp.float32),
        grid=(10,),
        in_specs=[spec, spec, spec, spec],
        out_specs=pl.BlockSpec((blk, 4, D_FEAT_K), lambda i: (i, 0, 0)),
    )(embed, *hops)


def kernel(embed, edge_index, trend):
    row = edge_index[0].astype(jnp.int32)
    col = edge_index[1].astype(jnp.int32)
    t = embed
    hops = []
    for _ in range(3):
        p0, p1 = _sc_hop(t, row, col, trend)
        t = _combine(p0, p1)
        hops.append(t)
    return _assemble(embed, hops)


# speedup vs baseline: 10.4269x; 1.0202x over previous
import functools

import jax
import jax.numpy as jnp
from jax import lax
from jax.experimental import pallas as pl
from jax.experimental.pallas import tpu as pltpu
from jax.experimental.pallas import tpu_sc as plsc

N_NODES_K = 10000
D_FEAT_K = 128
N_EDGES_K = 320000
CHUNK = 128
N_CHUNKS = N_EDGES_K // CHUNK
SC_CHUNKS = N_CHUNKS // 2
RPT = 632
RPT_LAST = N_NODES_K - 15 * RPT


def _hop_body(table, row, col, trend, out0, out1,
              ridx0, cidx0, tbuf0, rows0,
              ridx1, cidx1, tbuf1, rows1,
              ridx2, cidx2, tbuf2, rows2,
              acc, isem0, isem1, isem2,
              gsem0, gsem1, gsem2, ssem0, ssem1, ssem2):
    c = lax.axis_index("c")
    s = lax.axis_index("s")

    def _zfill(r, carry):
        for d in range(D_FEAT_K // 16):
            rows0[r, pl.ds(d * 16, 16)] = jnp.zeros((16,), jnp.float32)
        return carry

    lax.fori_loop(0, CHUNK, _zfill, 0)

    @pl.when(s < 15)
    def _():
        for q in range(4):
            pltpu.sync_copy(rows0, acc.at[pl.ds(s * RPT + q * CHUNK, CHUNK)])
        pltpu.sync_copy(rows0.at[pl.ds(0, RPT - 4 * CHUNK)],
                        acc.at[pl.ds(s * RPT + 4 * CHUNK, RPT - 4 * CHUNK)])

    @pl.when(s == 15)
    def _():
        for q in range(4):
            pltpu.sync_copy(rows0, acc.at[pl.ds(15 * RPT + q * CHUNK, CHUNK)])
        pltpu.sync_copy(rows0.at[pl.ds(0, RPT_LAST - 4 * CHUNK)],
                        acc.at[pl.ds(15 * RPT + 4 * CHUNK, RPT_LAST - 4 * CHUNK)])

    plsc.subcore_barrier()

    per = SC_CHUNKS // 16
    rem = SC_CHUNKS - 16 * per
    start = c * SC_CHUNKS + s * per + jnp.minimum(s, rem)
    n_my = jnp.where(s < rem, per + 1, per)

    bufs = ((ridx0, cidx0, tbuf0, rows0, isem0, gsem0, ssem0),
            (ridx1, cidx1, tbuf1, rows1, isem1, gsem1, ssem1),
            (ridx2, cidx2, tbuf2, rows2, isem2, gsem2, ssem2))

    def _fetch_idx(j, b):
        ridx, cidx, tbuf, _, isem, _, _ = bufs[b]

        @pl.when(j < n_my)
        def _():
            base = (start + j) * CHUNK
            pltpu.make_async_copy(row.at[pl.ds(base, CHUNK)], ridx, isem).start()
            pltpu.make_async_copy(col.at[pl.ds(base, CHUNK)], cidx, isem).start()
            pltpu.make_async_copy(trend.at[pl.ds(base, CHUNK)], tbuf, isem).start()

    def _start_gather(j, b):
        ridx, cidx, tbuf, rows, isem, gsem, _ = bufs[b]

        @pl.when(j < n_my)
        def _():
            pltpu.make_async_copy(row.at[pl.ds(0, CHUNK)], ridx, isem).wait()
            pltpu.make_async_copy(col.at[pl.ds(0, CHUNK)], cidx, isem).wait()
            pltpu.make_async_copy(trend.at[pl.ds(0, CHUNK)], tbuf, isem).wait()
            pltpu.make_async_copy(table.at[ridx], rows, gsem).start()

    def _wait_gather_scale(j, b):
        ridx, cidx, tbuf, rows, isem, gsem, _ = bufs[b]

        @pl.when(j < n_my)
        def _():
            pltpu.make_async_copy(table.at[ridx], rows, gsem).wait()

            def _scale(g, c2):
                sl16 = pl.ds(g * 16, 16)
                tv = tbuf[sl16]
                for j2 in range(16):
                    k = g * 16 + j2
                    t = tv[j2]
                    for d in range(D_FEAT_K // 16):
                        sl = pl.ds(d * 16, 16)
                        rows[k, sl] = rows[k, sl] * t
                return c2

            lax.fori_loop(0, CHUNK // 16, _scale, 0)

    def _start_scatter(j, b):
        _, cidx, _, rows, _, _, ssem = bufs[b]

        @pl.when(j < n_my)
        def _():
            pltpu.async_copy(rows, acc.at[cidx], ssem, add=True)

    def _wait_scatter(j, b):
        _, cidx, _, rows, _, _, ssem = bufs[b]

        @pl.when(jnp.logical_and(j >= 0, j < n_my))
        def _():
            pltpu.make_async_copy(rows, acc.at[cidx], ssem).wait()

    _fetch_idx(0, 0)
    _start_gather(0, 0)
    _fetch_idx(1, 1)

    def _outer(io, carry):
        for b in range(3):
            j = io * 3 + b
            nb = (b + 1) % 3
            pb = (b + 2) % 3
            _start_gather(j + 1, nb)
            _wait_gather_scale(j, b)
            _wait_scatter(j - 1, pb)
            _fetch_idx(j + 2, pb)
            _start_scatter(j, b)
        return carry

    lax.fori_loop(0, (per + 3) // 3 + 1, _outer, 0)
    plsc.subcore_barrier()

    @pl.when(jnp.logical_and(c == 0, s < 15))
    def _():
        sl = pl.ds(s * RPT, RPT)
        pltpu.sync_copy(acc.at[sl], out0.at[sl])

    @pl.when(jnp.logical_and(c == 0, s == 15))
    def _():
        sl = pl.ds(15 * RPT, RPT_LAST)
        pltpu.sync_copy(acc.at[sl], out0.at[sl])

    @pl.when(jnp.logical_and(c == 1, s < 15))
    def _():
        sl = pl.ds(s * RPT, RPT)
        pltpu.sync_copy(acc.at[sl], out1.at[sl])

    @pl.when(jnp.logical_and(c == 1, s == 15))
    def _():
        sl = pl.ds(15 * RPT, RPT_LAST)
        pltpu.sync_copy(acc.at[sl], out1.at[sl])


def _sc_hop(table, row, col, trend):
    mesh = plsc.VectorSubcoreMesh(core_axis_name="c", subcore_axis_name="s")
    f = functools.partial(
        pl.kernel,
        mesh=mesh,
        out_type=[
            jax.ShapeDtypeStruct((N_NODES_K, D_FEAT_K), jnp.float32),
            jax.ShapeDtypeStruct((N_NODES_K, D_FEAT_K), jnp.float32),
        ],
        scratch_types=(
            [pltpu.VMEM((CHUNK,), jnp.int32),
             pltpu.VMEM((CHUNK,), jnp.int32),
             pltpu.VMEM((CHUNK,), jnp.float32),
             pltpu.VMEM((CHUNK, D_FEAT_K), jnp.float32)] * 3
            + [pltpu.VMEM_SHARED((N_NODES_K, D_FEAT_K), jnp.float32)]
            + [pltpu.SemaphoreType.DMA] * 9
        ),
    )(_hop_body)
    return f(table, row, col, trend)


def _combine_body(a_ref, b_ref, o_ref):
    o_ref[...] = a_ref[...] + b_ref[...]


def _combine(p0, p1):
    blk = N_NODES_K // 10
    spec = pl.BlockSpec((blk, D_FEAT_K), lambda i: (i, 0))
    return pl.pallas_call(
        _combine_body,
        out_shape=jax.ShapeDtypeStruct((N_NODES_K, D_FEAT_K), jnp.float32),
        grid=(10,),
        in_specs=[spec, spec],
        out_specs=spec,
    )(p0, p1)


def _stack_body(e_ref, a1, a2, p0, p1, o_ref):
    o_ref[:, 0, :] = e_ref[...]
    o_ref[:, 1, :] = a1[...]
    o_ref[:, 2, :] = a2[...]
    o_ref[:, 3, :] = p0[...] + p1[...]


def _assemble(embed, t1, t2, p0, p1):
    blk = N_NODES_K // 10
    spec = pl.BlockSpec((blk, D_FEAT_K), lambda i: (i, 0))
    return pl.pallas_call(
        _stack_body,
        out_shape=jax.ShapeDtypeStruct((N_NODES_K, 4, D_FEAT_K), jnp.float32),
        grid=(10,),
        in_specs=[spec, spec, spec, spec, spec],
        out_specs=pl.BlockSpec((blk, 4, D_FEAT_K), lambda i: (i, 0, 0)),
    )(embed, t1, t2, p0, p1)


def kernel(embed, edge_index, trend):
    row = edge_index[0].astype(jnp.int32)
    col = edge_index[1].astype(jnp.int32)
    p0, p1 = _sc_hop(embed, row, col, trend)
    t1 = _combine(p0, p1)
    p0, p1 = _sc_hop(t1, row, col, trend)
    t2 = _combine(p0, p1)
    p0, p1 = _sc_hop(t2, row, col, trend)
    return _assemble(embed, t1, t2, p0, p1)


# speedup vs baseline: 10.5559x; 1.0124x over previous
import functools

import jax
import jax.numpy as jnp
from jax import lax
from jax.experimental import pallas as pl
from jax.experimental.pallas import tpu as pltpu
from jax.experimental.pallas import tpu_sc as plsc

N_NODES_K = 10000
D_FEAT_K = 128
N_EDGES_K = 320000
CHUNK = 128
N_CHUNKS = N_EDGES_K // CHUNK
SC_CHUNKS = N_CHUNKS // 2
RPT = 632
RPT_LAST = N_NODES_K - 15 * RPT


def _hop_body(table, epk, out0, out1,
              ebuf0, rows0, ebuf1, rows1, ebuf2, rows2,
              acc, isem0, isem1, isem2,
              gsem0, gsem1, gsem2, ssem0, ssem1, ssem2):
    c = lax.axis_index("c")
    s = lax.axis_index("s")

    def _zfill(r, carry):
        for d in range(D_FEAT_K // 16):
            rows0[r, pl.ds(d * 16, 16)] = jnp.zeros((16,), jnp.float32)
        return carry

    lax.fori_loop(0, CHUNK, _zfill, 0)

    @pl.when(s < 15)
    def _():
        for q in range(4):
            pltpu.sync_copy(rows0, acc.at[pl.ds(s * RPT + q * CHUNK, CHUNK)])
        pltpu.sync_copy(rows0.at[pl.ds(0, RPT - 4 * CHUNK)],
                        acc.at[pl.ds(s * RPT + 4 * CHUNK, RPT - 4 * CHUNK)])

    @pl.when(s == 15)
    def _():
        for q in range(4):
            pltpu.sync_copy(rows0, acc.at[pl.ds(15 * RPT + q * CHUNK, CHUNK)])
        pltpu.sync_copy(rows0.at[pl.ds(0, RPT_LAST - 4 * CHUNK)],
                        acc.at[pl.ds(15 * RPT + 4 * CHUNK, RPT_LAST - 4 * CHUNK)])

    plsc.subcore_barrier()

    per = SC_CHUNKS // 16
    rem = SC_CHUNKS - 16 * per
    start = c * SC_CHUNKS + s * per + jnp.minimum(s, rem)
    n_my = jnp.where(s < rem, per + 1, per)

    bufs = ((ebuf0, rows0, isem0, gsem0, ssem0),
            (ebuf1, rows1, isem1, gsem1, ssem1),
            (ebuf2, rows2, isem2, gsem2, ssem2))

    def _fetch_idx(j, b):
        ebuf, _, isem, _, _ = bufs[b]

        @pl.when(j < n_my)
        def _():
            pltpu.make_async_copy(epk.at[start + j], ebuf, isem).start()

    def _start_gather(j, b):
        ebuf, rows, isem, gsem, _ = bufs[b]

        @pl.when(j < n_my)
        def _():
            pltpu.make_async_copy(epk.at[0], ebuf, isem).wait()
            pltpu.make_async_copy(table.at[ebuf.at[0]], rows, gsem).start()

    def _wait_gather_scale(j, b):
        ebuf, rows, isem, gsem, _ = bufs[b]

        @pl.when(j < n_my)
        def _():
            pltpu.make_async_copy(table.at[ebuf.at[0]], rows, gsem).wait()

            def _scale(g, c2):
                sl16 = pl.ds(g * 16, 16)
                tv = lax.bitcast_convert_type(ebuf[2, sl16], jnp.float32)
                for j2 in range(16):
                    k = g * 16 + j2
                    t = tv[j2]
                    for d in range(D_FEAT_K // 16):
                        sl = pl.ds(d * 16, 16)
                        rows[k, sl] = rows[k, sl] * t
                return c2

            lax.fori_loop(0, CHUNK // 16, _scale, 0)

    def _start_scatter(j, b):
        ebuf, rows, _, _, ssem = bufs[b]

        @pl.when(j < n_my)
        def _():
            pltpu.async_copy(rows, acc.at[ebuf.at[1]], ssem, add=True)

    def _wait_scatter(j, b):
        ebuf, rows, _, _, ssem = bufs[b]

        @pl.when(jnp.logical_and(j >= 0, j < n_my))
        def _():
            pltpu.make_async_copy(rows, acc.at[ebuf.at[1]], ssem).wait()

    _fetch_idx(0, 0)
    _start_gather(0, 0)
    _fetch_idx(1, 1)

    def _outer(io, carry):
        for b in range(3):
            j = io * 3 + b
            nb = (b + 1) % 3
            pb = (b + 2) % 3
            _start_gather(j + 1, nb)
            _wait_gather_scale(j, b)
            _wait_scatter(j - 1, pb)
            _fetch_idx(j + 2, pb)
            _start_scatter(j, b)
        return carry

    lax.fori_loop(0, (per + 3) // 3 + 1, _outer, 0)
    plsc.subcore_barrier()

    @pl.when(jnp.logical_and(c == 0, s < 15))
    def _():
        sl = pl.ds(s * RPT, RPT)
        pltpu.sync_copy(acc.at[sl], out0.at[sl])

    @pl.when(jnp.logical_and(c == 0, s == 15))
    def _():
        sl = pl.ds(15 * RPT, RPT_LAST)
        pltpu.sync_copy(acc.at[sl], out0.at[sl])

    @pl.when(jnp.logical_and(c == 1, s < 15))
    def _():
        sl = pl.ds(s * RPT, RPT)
        pltpu.sync_copy(acc.at[sl], out1.at[sl])

    @pl.when(jnp.logical_and(c == 1, s == 15))
    def _():
        sl = pl.ds(15 * RPT, RPT_LAST)
        pltpu.sync_copy(acc.at[sl], out1.at[sl])


def _sc_hop(table, epk):
    mesh = plsc.VectorSubcoreMesh(core_axis_name="c", subcore_axis_name="s")
    f = functools.partial(
        pl.kernel,
        mesh=mesh,
        out_type=[
            jax.ShapeDtypeStruct((N_NODES_K, D_FEAT_K), jnp.float32),
            jax.ShapeDtypeStruct((N_NODES_K, D_FEAT_K), jnp.float32),
        ],
        scratch_types=(
            [pltpu.VMEM((3, CHUNK), jnp.int32),
             pltpu.VMEM((CHUNK, D_FEAT_K), jnp.float32)] * 3
            + [pltpu.VMEM_SHARED((N_NODES_K, D_FEAT_K), jnp.float32)]
            + [pltpu.SemaphoreType.DMA] * 9
        ),
    )(_hop_body)
    return f(table, epk)


def _combine_body(a_ref, b_ref, o_ref):
    o_ref[...] = a_ref[...] + b_ref[...]


def _combine(p0, p1):
    blk = N_NODES_K // 10
    spec = pl.BlockSpec((blk, D_FEAT_K), lambda i: (i, 0))
    return pl.pallas_call(
        _combine_body,
        out_shape=jax.ShapeDtypeStruct((N_NODES_K, D_FEAT_K), jnp.float32),
        grid=(10,),
        in_specs=[spec, spec],
        out_specs=spec,
    )(p0, p1)


def _stack_body(e_ref, a1, a2, p0, p1, o_ref):
    o_ref[:, 0, :] = e_ref[...]
    o_ref[:, 1, :] = a1[...]
    o_ref[:, 2, :] = a2[...]
    o_ref[:, 3, :] = p0[...] + p1[...]


def _assemble(embed, t1, t2, p0, p1):
    blk = N_NODES_K // 10
    spec = pl.BlockSpec((blk, D_FEAT_K), lambda i: (i, 0))
    return pl.pallas_call(
        _stack_body,
        out_shape=jax.ShapeDtypeStruct((N_NODES_K, 4, D_FEAT_K), jnp.float32),
        grid=(10,),
        in_specs=[spec, spec, spec, spec, spec],
        out_specs=pl.BlockSpec((blk, 4, D_FEAT_K), lambda i: (i, 0, 0)),
    )(embed, t1, t2, p0, p1)


def kernel(embed, edge_index, trend):
    row = edge_index[0].astype(jnp.int32).reshape(N_CHUNKS, CHUNK)
    col = edge_index[1].astype(jnp.int32).reshape(N_CHUNKS, CHUNK)
    tbits = lax.bitcast_convert_type(trend, jnp.int32).reshape(N_CHUNKS, CHUNK)
    epk = jnp.stack([row, col, tbits], axis=1)
    p0, p1 = _sc_hop(embed, epk)
    t1 = _combine(p0, p1)
    p0, p1 = _sc_hop(t1, epk)
    t2 = _combine(p0, p1)
    p0, p1 = _sc_hop(t2, epk)
    return _assemble(embed, t1, t2, p0, p1)


# speedup vs baseline: 10.7257x; 1.0161x over previous
import functools

import jax
import jax.numpy as jnp
from jax import lax
from jax.experimental import pallas as pl
from jax.experimental.pallas import tpu as pltpu
from jax.experimental.pallas import tpu_sc as plsc

N_NODES_K = 10000
D_FEAT_K = 128
N_EDGES_K = 320000
CHUNK = 128
N_CHUNKS = N_EDGES_K // CHUNK
SC_CHUNKS = N_CHUNKS // 2
RPT = 632
RPT_LAST = N_NODES_K - 15 * RPT


def _hop_body(table, epk, out0, out1,
              ebuf0, rows0, ebuf1, rows1, ebuf2, rows2,
              acc, isem0, isem1, isem2,
              gsem0, gsem1, gsem2, ssem0, ssem1, ssem2):
    c = lax.axis_index("c")
    s = lax.axis_index("s")

    per = SC_CHUNKS // 16
    rem = SC_CHUNKS - 16 * per
    start = c * SC_CHUNKS + s * per + jnp.minimum(s, rem)
    n_my = jnp.where(s < rem, per + 1, per)

    bufs = ((ebuf0, rows0, isem0, gsem0, ssem0),
            (ebuf1, rows1, isem1, gsem1, ssem1),
            (ebuf2, rows2, isem2, gsem2, ssem2))

    def _fetch_idx(j, b):
        ebuf, _, isem, _, _ = bufs[b]

        @pl.when(j < n_my)
        def _():
            pltpu.make_async_copy(epk.at[start + j], ebuf, isem).start()

    def _start_gather(j, b):
        ebuf, rows, isem, gsem, _ = bufs[b]

        @pl.when(j < n_my)
        def _():
            pltpu.make_async_copy(epk.at[0], ebuf, isem).wait()
            pltpu.make_async_copy(table.at[ebuf.at[0]], rows, gsem).start()

    def _wait_gather_scale(j, b):
        ebuf, rows, isem, gsem, _ = bufs[b]

        @pl.when(j < n_my)
        def _():
            pltpu.make_async_copy(table.at[ebuf.at[0]], rows, gsem).wait()

            def _scale(g, c2):
                sl16 = pl.ds(g * 16, 16)
                tv = lax.bitcast_convert_type(ebuf[2, sl16], jnp.float32)
                for j2 in range(16):
                    k = g * 16 + j2
                    t = tv[j2]
                    for d in range(D_FEAT_K // 16):
                        sl = pl.ds(d * 16, 16)
                        rows[k, sl] = rows[k, sl] * t
                return c2

            lax.fori_loop(0, CHUNK // 16, _scale, 0)

    def _start_scatter(j, b):
        ebuf, rows, _, _, ssem = bufs[b]

        @pl.when(j < n_my)
        def _():
            pltpu.async_copy(rows, acc.at[ebuf.at[1]], ssem, add=True)

    def _wait_scatter(j, b):
        ebuf, rows, _, _, ssem = bufs[b]

        @pl.when(jnp.logical_and(j >= 0, j < n_my))
        def _():
            pltpu.make_async_copy(rows, acc.at[ebuf.at[1]], ssem).wait()

    _fetch_idx(0, 0)
    _start_gather(0, 0)
    _fetch_idx(1, 1)

    def _zfill(r, carry):
        for d in range(D_FEAT_K // 16):
            rows2[r, pl.ds(d * 16, 16)] = jnp.zeros((16,), jnp.float32)
        return carry

    lax.fori_loop(0, CHUNK, _zfill, 0)

    @pl.when(s < 15)
    def _():
        for q in range(4):
            pltpu.sync_copy(rows2, acc.at[pl.ds(s * RPT + q * CHUNK, CHUNK)])
        pltpu.sync_copy(rows2.at[pl.ds(0, RPT - 4 * CHUNK)],
                        acc.at[pl.ds(s * RPT + 4 * CHUNK, RPT - 4 * CHUNK)])

    @pl.when(s == 15)
    def _():
        for q in range(4):
            pltpu.sync_copy(rows2, acc.at[pl.ds(15 * RPT + q * CHUNK, CHUNK)])
        pltpu.sync_copy(rows2.at[pl.ds(0, RPT_LAST - 4 * CHUNK)],
                        acc.at[pl.ds(15 * RPT + 4 * CHUNK, RPT_LAST - 4 * CHUNK)])

    plsc.subcore_barrier()

    def _outer(io, carry):
        for b in range(3):
            j = io * 3 + b
            nb = (b + 1) % 3
            pb = (b + 2) % 3
            _start_gather(j + 1, nb)
            _wait_gather_scale(j, b)
            _wait_scatter(j - 1, pb)
            _fetch_idx(j + 2, pb)
            _start_scatter(j, b)
        return carry

    lax.fori_loop(0, (per + 3) // 3 + 1, _outer, 0)
    plsc.subcore_barrier()

    @pl.when(jnp.logical_and(c == 0, s < 15))
    def _():
        sl = pl.ds(s * RPT, RPT)
        pltpu.sync_copy(acc.at[sl], out0.at[sl])

    @pl.when(jnp.logical_and(c == 0, s == 15))
    def _():
        sl = pl.ds(15 * RPT, RPT_LAST)
        pltpu.sync_copy(acc.at[sl], out0.at[sl])

    @pl.when(jnp.logical_and(c == 1, s < 15))
    def _():
        sl = pl.ds(s * RPT, RPT)
        pltpu.sync_copy(acc.at[sl], out1.at[sl])

    @pl.when(jnp.logical_and(c == 1, s == 15))
    def _():
        sl = pl.ds(15 * RPT, RPT_LAST)
        pltpu.sync_copy(acc.at[sl], out1.at[sl])


def _sc_hop(table, epk):
    mesh = plsc.VectorSubcoreMesh(core_axis_name="c", subcore_axis_name="s")
    f = functools.partial(
        pl.kernel,
        mesh=mesh,
        out_type=[
            jax.ShapeDtypeStruct((N_NODES_K, D_FEAT_K), jnp.float32),
            jax.ShapeDtypeStruct((N_NODES_K, D_FEAT_K), jnp.float32),
        ],
        scratch_types=(
            [pltpu.VMEM((3, CHUNK), jnp.int32),
             pltpu.VMEM((CHUNK, D_FEAT_K), jnp.float32)] * 3
            + [pltpu.VMEM_SHARED((N_NODES_K, D_FEAT_K), jnp.float32)]
            + [pltpu.SemaphoreType.DMA] * 9
        ),
    )(_hop_body)
    return f(table, epk)


def _combine_body(a_ref, b_ref, o_ref):
    o_ref[...] = a_ref[...] + b_ref[...]


def _combine(p0, p1):
    blk = N_NODES_K // 10
    spec = pl.BlockSpec((blk, D_FEAT_K), lambda i: (i, 0))
    return pl.pallas_call(
        _combine_body,
        out_shape=jax.ShapeDtypeStruct((N_NODES_K, D_FEAT_K), jnp.float32),
        grid=(10,),
        in_specs=[spec, spec],
        out_specs=spec,
    )(p0, p1)


def _stack_body(e_ref, a1, a2, p0, p1, o_ref):
    o_ref[:, 0, :] = e_ref[...]
    o_ref[:, 1, :] = a1[...]
    o_ref[:, 2, :] = a2[...]
    o_ref[:, 3, :] = p0[...] + p1[...]


def _assemble(embed, t1, t2, p0, p1):
    blk = N_NODES_K // 10
    spec = pl.BlockSpec((blk, D_FEAT_K), lambda i: (i, 0))
    return pl.pallas_call(
        _stack_body,
        out_shape=jax.ShapeDtypeStruct((N_NODES_K, 4, D_FEAT_K), jnp.float32),
        grid=(10,),
        in_specs=[spec, spec, spec, spec, spec],
        out_specs=pl.BlockSpec((blk, 4, D_FEAT_K), lambda i: (i, 0, 0)),
    )(embed, t1, t2, p0, p1)


def kernel(embed, edge_index, trend):
    row = edge_index[0].astype(jnp.int32).reshape(N_CHUNKS, CHUNK)
    col = edge_index[1].astype(jnp.int32).reshape(N_CHUNKS, CHUNK)
    tbits = lax.bitcast_convert_type(trend, jnp.int32).reshape(N_CHUNKS, CHUNK)
    epk = jnp.stack([row, col, tbits], axis=1)
    p0, p1 = _sc_hop(embed, epk)
    t1 = _combine(p0, p1)
    p0, p1 = _sc_hop(t1, epk)
    t2 = _combine(p0, p1)
    p0, p1 = _sc_hop(t2, epk)
    return _assemble(embed, t1, t2, p0, p1)
